# baseline (speedup 1.0000x reference)
import functools

import jax
import jax.numpy as jnp
from jax import lax
from jax.experimental import pallas as pl
from jax.experimental.pallas import tpu as pltpu
from jax.experimental.pallas import tpu_sc as plsc

N = 10000
E = 320000
F = 128
F3 = 128
C = 40
NP = 10240

_info = plsc.get_sparse_core_info()
NC = _info.num_cores
NS = _info.num_subcores
NW = NC * NS
EPT = E // NW
CH = 80
NCHUNK = EPT // CH
RPT = NP // NS


def _make_aggregate(width, with_deg):
    out_type = [jax.ShapeDtypeStruct((NC, NP, width), jnp.float32)]
    scratch = [
        pltpu.VMEM((2, CH), jnp.int32),
        pltpu.VMEM((4, CH), jnp.int32),
        pltpu.VMEM((CH, width), jnp.float32),
        pltpu.VMEM((CH, width), jnp.float32),
        pltpu.VMEM_SHARED((NP, width), jnp.float32),
        pltpu.SemaphoreType.DMA,
        pltpu.SemaphoreType.DMA,
        pltpu.SemaphoreType.DMA,
        pltpu.SemaphoreType.DMA,
        pltpu.SemaphoreType.DMA,
        pltpu.SemaphoreType.DMA,
    ]
    if with_deg:
        out_type.append(jax.ShapeDtypeStruct((NC, NP), jnp.float32))
        scratch += [
            pltpu.VMEM((CH,), jnp.float32),
            pltpu.VMEM_SHARED((NP,), jnp.float32),
        ]

    def body(*refs):
        if with_deg:
            (src_hbm, dst_hbm, h_hbm, zeros_hbm, zerosd_hbm,
             out_hbm, deg_hbm,
             sidx, didx, buf0, buf1, acc,
             sem0, sem1, semi0, semi1, semsc0, semsc1,
             ones_v, degacc) = refs
        else:
            (src_hbm, dst_hbm, h_hbm, zeros_hbm,
             out_hbm,
             sidx, didx, buf0, buf1, acc,
             sem0, sem1, semi0, semi1, semsc0, semsc1) = refs

        c = lax.axis_index("c")
        s = lax.axis_index("s")
        wid = c * NS + s
        r0 = s * RPT
        pltpu.sync_copy(zeros_hbm.at[pl.ds(r0, RPT)], acc.at[pl.ds(r0, RPT)])
        if with_deg:
            pltpu.sync_copy(zerosd_hbm.at[pl.ds(r0, RPT)],
                            degacc.at[pl.ds(r0, RPT)])
            for j in range(CH // 16):
                ones_v[pl.ds(j * 16, 16)] = jnp.full((16,), 1.0, jnp.float32)

        def idx_load(i, sslot, dslot, semi):
            pltpu.async_copy(src_hbm.at[wid, i], sidx.at[sslot], semi)
            pltpu.async_copy(dst_hbm.at[wid, i], didx.at[dslot], semi)

        def idx_wait(sslot, semi):
            pltpu.make_async_copy(src_hbm.at[wid, 0], sidx.at[sslot], semi).wait()
            pltpu.make_async_copy(dst_hbm.at[wid, 0], didx.at[0], semi).wait()

        def gather(sslot, buf, sem):
            pltpu.async_copy(h_hbm.at[sidx.at[sslot]], buf, sem)

        def gather_wait(buf, sem):
            pltpu.make_async_copy(h_hbm.at[sidx.at[0]], buf, sem).wait()

        def scatter_start(buf, dslot, semsc):
            pltpu.async_copy(buf, acc.at[didx.at[dslot]], semsc, add=True)
            if with_deg:
                pltpu.sync_copy(ones_v, degacc.at[didx.at[dslot]], add=True)

        def scatter_wait(buf, semsc):
            pltpu.make_async_copy(buf, acc.at[didx.at[0]], semsc).wait()

        idx_load(0, 0, 0, semi0)
        idx_load(1, 1, 1, semi1)
        idx_wait(0, semi0)
        gather(0, buf0, sem0)
        plsc.subcore_barrier()

        def step(j, carry):
            i0 = 2 * j
            jm = lax.rem(j, 2)
            d_cur = 2 * jm
            d_nxt = 2 - 2 * jm
            idx_wait(1, semi1)
            gather_wait(buf0, sem0)

            @pl.when(j > 0)
            def _():
                scatter_wait(buf1, semsc1)

            scatter_start(buf0, d_cur, semsc0)
            gather(1, buf1, sem1)
            idx_load(i0 + 2, 0, d_nxt, semi0)

            gather_wait(buf1, sem1)
            scatter_wait(buf0, semsc0)
            scatter_start(buf1, d_cur + 1, semsc1)
            idx_wait(0, semi0)
            gather(0, buf0, sem0)

            @pl.when(i0 + 3 < NCHUNK)
            def _():
                idx_load(i0 + 3, 1, d_nxt + 1, semi1)

            return carry

        lax.fori_loop(0, (NCHUNK - 1) // 2, step, 0)
        gather_wait(buf0, sem0)
        scatter_wait(buf1, semsc1)
        scatter_start(buf0, 2 * (((NCHUNK - 1) // 2) % 2), semsc0)
        scatter_wait(buf0, semsc0)
        plsc.subcore_barrier()
        pltpu.sync_copy(acc.at[pl.ds(r0, RPT)], out_hbm.at[c, pl.ds(r0, RPT)])
        if with_deg:
            pltpu.sync_copy(degacc.at[pl.ds(r0, RPT)],
                            deg_hbm.at[c, pl.ds(r0, RPT)])

    return pl.kernel(
        body,
        out_type=out_type if len(out_type) > 1 else out_type[0],
        mesh=plsc.VectorSubcoreMesh(core_axis_name="c", subcore_axis_name="s"),
        scratch_types=scratch,
    )


_sc_aggregate_deg = _make_aggregate(F, True)
_sc_aggregate = _make_aggregate(F, False)
_sc_aggregate64 = _make_aggregate(F3, False)


BLK = 512
_row = lambda i: (i, 0)
_rep = lambda i: (0, 0)


def _mean(p0, p1, d0, d1):
    deg = jnp.maximum(d0[...] + d1[...], 1.0)
    return (p0[...] + p1[...]) / deg


def _dense_body(p0, p1, d0, d1, h, wl, wr, b, o, *, act):
    acc = jnp.dot(_mean(p0, p1, d0, d1), wl[...],
                  preferred_element_type=jnp.float32)
    acc = acc + jnp.dot(h[...], wr[...], preferred_element_type=jnp.float32)
    acc = acc + b[...]
    if act:
        acc = jnp.maximum(acc, 0.0)
    o[...] = acc


def _dense(p, d0, d1, h, Wl, Wr, b):
    return pl.pallas_call(
        functools.partial(_dense_body, act=True),
        grid=(NP // BLK,),
        in_specs=[
            pl.BlockSpec((BLK, F), _row),
            pl.BlockSpec((BLK, F), _row),
            pl.BlockSpec((BLK, 1), _row),
            pl.BlockSpec((BLK, 1), _row),
            pl.BlockSpec((BLK, F), _row),
            pl.BlockSpec((F, F), _rep),
            pl.BlockSpec((F, F), _rep),
            pl.BlockSpec((1, F), _rep),
        ],
        out_specs=pl.BlockSpec((BLK, F), _row),
        out_shape=jax.ShapeDtypeStruct((NP, F), jnp.float32),
    )(p[0], p[1], d0, d1, h, Wl, Wr, b)


def _dense_mid_body(p0, p1, d0, d1, h, wl, wr, b, w3l, o, oy, *, act=True):
    acc = jnp.dot(_mean(p0, p1, d0, d1), wl[...],
                  preferred_element_type=jnp.float32)
    acc = acc + jnp.dot(h[...], wr[...], preferred_element_type=jnp.float32)
    h2 = jnp.maximum(acc + b[...], 0.0)
    o[...] = h2
    oy[...] = jnp.dot(h2, w3l[...], preferred_element_type=jnp.float32)


def _dense_mid(p, d0, d1, h, Wl, Wr, b, W3l):
    return pl.pallas_call(
        _dense_mid_body,
        grid=(NP // BLK,),
        in_specs=[
            pl.BlockSpec((BLK, F), _row),
            pl.BlockSpec((BLK, F), _row),
            pl.BlockSpec((BLK, 1), _row),
            pl.BlockSpec((BLK, 1), _row),
            pl.BlockSpec((BLK, F), _row),
            pl.BlockSpec((F, F), _rep),
            pl.BlockSpec((F, F), _rep),
            pl.BlockSpec((1, F), _rep),
            pl.BlockSpec((F, F3), _rep),
        ],
        out_specs=[pl.BlockSpec((BLK, F), _row), pl.BlockSpec((BLK, F3), _row)],
        out_shape=[jax.ShapeDtypeStruct((NP, F), jnp.float32),
                   jax.ShapeDtypeStruct((NP, F3), jnp.float32)],
    )(p[0], p[1], d0, d1, h, Wl, Wr, b, W3l)


def _final_body(py0, py1, d0, d1, h, wr, b, o):
    acc = _mean(py0, py1, d0, d1)
    acc = acc + jnp.dot(h[...], wr[...], preferred_element_type=jnp.float32)
    o[...] = acc + b[...]


def _final(py, d0, d1, h, W3r, b3):
    return pl.pallas_call(
        _final_body,
        grid=(NP // BLK,),
        in_specs=[
            pl.BlockSpec((BLK, F3), _row),
            pl.BlockSpec((BLK, F3), _row),
            pl.BlockSpec((BLK, 1), _row),
            pl.BlockSpec((BLK, 1), _row),
            pl.BlockSpec((BLK, F), _row),
            pl.BlockSpec((F, F3), _rep),
            pl.BlockSpec((1, F3), _rep),
        ],
        out_specs=pl.BlockSpec((BLK, F3), _row),
        out_shape=jax.ShapeDtypeStruct((NP, F3), jnp.float32),
    )(py[0], py[1], d0, d1, h, W3r, b3)


def _pad64(w):
    return jnp.pad(w, ((0, 0), (0, F3 - w.shape[1])))


def kernel(x, edge_index, W1l, W1r, b1, W2l, W2r, b2, W3l, W3r, b3):
    src3 = edge_index[0].reshape(NW, NCHUNK, CH)
    dst3 = edge_index[1].reshape(NW, NCHUNK, CH)
    zeros2d = jnp.zeros((NP, F), jnp.float32)
    zeros64 = jnp.zeros((NP, F3), jnp.float32)
    zerosd = jnp.zeros((NP,), jnp.float32)
    xp = jnp.pad(x, ((0, NP - N), (0, 0)))

    p, degp = _sc_aggregate_deg(src3, dst3, xp, zeros2d, zerosd)
    d0 = degp[0].reshape(NP, 1)
    d1 = degp[1].reshape(NP, 1)
    h = _dense(p, d0, d1, xp, W1l, W1r, b1.reshape(1, F))

    p = _sc_aggregate(src3, dst3, h, zeros2d)
    h, y = _dense_mid(p, d0, d1, h, W2l, W2r, b2.reshape(1, F), _pad64(W3l))

    py = _sc_aggregate64(src3, dst3, y, zeros64)
    out = _final(py, d0, d1, h, _pad64(W3r),
                 jnp.pad(b3, (0, F3 - C)).reshape(1, F3))
    return out[:N, :C]


# speedup vs baseline: 1.0267x; 1.0267x over previous
import functools

import jax
import jax.numpy as jnp
from jax import lax
from jax.experimental import pallas as pl
from jax.experimental.pallas import tpu as pltpu
from jax.experimental.pallas import tpu_sc as plsc

N = 10000
E = 320000
F = 128
F3 = 128
C = 40
NP = 10240

_info = plsc.get_sparse_core_info()
NC = _info.num_cores
NS = _info.num_subcores
NW = NC * NS
EPT = E // NW
CH = 80
NCHUNK = EPT // CH
RPT = NP // NS


def _make_aggregate(width, with_deg):
    out_type = [jax.ShapeDtypeStruct((NC, NP, width), jnp.float32)]
    scratch = [
        pltpu.VMEM((2, CH), jnp.int32),
        pltpu.VMEM((2, CH), jnp.int32),
        pltpu.VMEM((CH, width), jnp.float32),
        pltpu.VMEM((CH, width), jnp.float32),
        pltpu.VMEM_SHARED((NP, width), jnp.float32),
        pltpu.SemaphoreType.DMA,
        pltpu.SemaphoreType.DMA,
        pltpu.SemaphoreType.DMA,
        pltpu.SemaphoreType.DMA,
        pltpu.SemaphoreType.DMA,
        pltpu.SemaphoreType.DMA,
    ]
    if with_deg:
        out_type.append(jax.ShapeDtypeStruct((NC, NP), jnp.float32))
        scratch += [
            pltpu.VMEM((CH,), jnp.float32),
            pltpu.VMEM_SHARED((NP,), jnp.float32),
        ]

    def body(*refs):
        if with_deg:
            (src_hbm, dst_hbm, h_hbm, zeros_hbm, zerosd_hbm,
             out_hbm, deg_hbm,
             sidx, didx, buf0, buf1, acc,
             sem0, sem1, semsi0, semsi1, semdi0, semdi1,
             ones_v, degacc) = refs
        else:
            (src_hbm, dst_hbm, h_hbm, zeros_hbm,
             out_hbm,
             sidx, didx, buf0, buf1, acc,
             sem0, sem1, semsi0, semsi1, semdi0, semdi1) = refs

        c = lax.axis_index("c")
        s = lax.axis_index("s")
        wid = c * NS + s
        r0 = s * RPT
        pltpu.sync_copy(zeros_hbm.at[pl.ds(r0, RPT)], acc.at[pl.ds(r0, RPT)])
        if with_deg:
            pltpu.sync_copy(zerosd_hbm.at[pl.ds(r0, RPT)],
                            degacc.at[pl.ds(r0, RPT)])
            for j in range(CH // 16):
                ones_v[pl.ds(j * 16, 16)] = jnp.full((16,), 1.0, jnp.float32)

        def sidx_load(i, slot, sem):
            pltpu.async_copy(src_hbm.at[wid, i], sidx.at[slot], sem)

        def sidx_wait(slot, sem):
            pltpu.make_async_copy(src_hbm.at[wid, 0], sidx.at[slot], sem).wait()

        def didx_load(i, slot, sem):
            pltpu.async_copy(dst_hbm.at[wid, i], didx.at[slot], sem)

        def didx_wait(slot, sem):
            pltpu.make_async_copy(dst_hbm.at[wid, 0], didx.at[slot], sem).wait()

        def gather(slot, buf, sem):
            pltpu.async_copy(h_hbm.at[sidx.at[slot]], buf, sem)

        def gather_wait(buf, sem):
            pltpu.make_async_copy(h_hbm.at[sidx.at[0]], buf, sem).wait()

        def scatter(buf, slot):
            pltpu.sync_copy(buf, acc.at[didx.at[slot]], add=True)
            if with_deg:
                pltpu.sync_copy(ones_v, degacc.at[didx.at[slot]], add=True)

        sidx_load(0, 0, semsi0)
        didx_load(0, 0, semdi0)
        sidx_load(1, 1, semsi1)
        didx_load(1, 1, semdi1)
        sidx_wait(0, semsi0)
        gather(0, buf0, sem0)
        plsc.subcore_barrier()

        def step(j, carry):
            i0 = 2 * j
            sidx_wait(1, semsi1)
            gather_wait(buf0, sem0)
            gather(1, buf1, sem1)
            sidx_load(i0 + 2, 0, semsi0)
            didx_wait(0, semdi0)
            scatter(buf0, 0)
            didx_load(i0 + 2, 0, semdi0)
            sidx_wait(0, semsi0)
            gather_wait(buf1, sem1)
            gather(0, buf0, sem0)

            @pl.when(i0 + 3 < NCHUNK)
            def _():
                sidx_load(i0 + 3, 1, semsi1)

            didx_wait(1, semdi1)
            scatter(buf1, 1)

            @pl.when(i0 + 3 < NCHUNK)
            def _():
                didx_load(i0 + 3, 1, semdi1)

            return carry

        lax.fori_loop(0, (NCHUNK - 1) // 2, step, 0)
        gather_wait(buf0, sem0)
        didx_wait(0, semdi0)
        scatter(buf0, 0)
        plsc.subcore_barrier()
        pltpu.sync_copy(acc.at[pl.ds(r0, RPT)], out_hbm.at[c, pl.ds(r0, RPT)])
        if with_deg:
            pltpu.sync_copy(degacc.at[pl.ds(r0, RPT)],
                            deg_hbm.at[c, pl.ds(r0, RPT)])

    return pl.kernel(
        body,
        out_type=out_type if len(out_type) > 1 else out_type[0],
        mesh=plsc.VectorSubcoreMesh(core_axis_name="c", subcore_axis_name="s"),
        scratch_types=scratch,
    )


_sc_aggregate_deg = _make_aggregate(F, True)
_sc_aggregate = _make_aggregate(F, False)
_sc_aggregate64 = _make_aggregate(F3, False)


BLK = 512
_row = lambda i: (i, 0)
_rep = lambda i: (0, 0)


def _mean(p0, p1, d0, d1):
    deg = jnp.maximum(d0[...] + d1[...], 1.0)
    return (p0[...] + p1[...]) / deg


def _dense_body(p0, p1, d0, d1, h, wl, wr, b, o, *, act):
    acc = jnp.dot(_mean(p0, p1, d0, d1), wl[...],
                  preferred_element_type=jnp.float32)
    acc = acc + jnp.dot(h[...], wr[...], preferred_element_type=jnp.float32)
    acc = acc + b[...]
    if act:
        acc = jnp.maximum(acc, 0.0)
    o[...] = acc


def _dense(p, d0, d1, h, Wl, Wr, b):
    return pl.pallas_call(
        functools.partial(_dense_body, act=True),
        grid=(NP // BLK,),
        in_specs=[
            pl.BlockSpec((BLK, F), _row),
            pl.BlockSpec((BLK, F), _row),
            pl.BlockSpec((BLK, 1), _row),
            pl.BlockSpec((BLK, 1), _row),
            pl.BlockSpec((BLK, F), _row),
            pl.BlockSpec((F, F), _rep),
            pl.BlockSpec((F, F), _rep),
            pl.BlockSpec((1, F), _rep),
        ],
        out_specs=pl.BlockSpec((BLK, F), _row),
        out_shape=jax.ShapeDtypeStruct((NP, F), jnp.float32),
    )(p[0], p[1], d0, d1, h, Wl, Wr, b)


def _dense_mid_body(p0, p1, d0, d1, h, wl, wr, b, w3l, o, oy, *, act=True):
    acc = jnp.dot(_mean(p0, p1, d0, d1), wl[...],
                  preferred_element_type=jnp.float32)
    acc = acc + jnp.dot(h[...], wr[...], preferred_element_type=jnp.float32)
    h2 = jnp.maximum(acc + b[...], 0.0)
    o[...] = h2
    oy[...] = jnp.dot(h2, w3l[...], preferred_element_type=jnp.float32)


def _dense_mid(p, d0, d1, h, Wl, Wr, b, W3l):
    return pl.pallas_call(
        _dense_mid_body,
        grid=(NP // BLK,),
        in_specs=[
            pl.BlockSpec((BLK, F), _row),
            pl.BlockSpec((BLK, F), _row),
            pl.BlockSpec((BLK, 1), _row),
            pl.BlockSpec((BLK, 1), _row),
            pl.BlockSpec((BLK, F), _row),
            pl.BlockSpec((F, F), _rep),
            pl.BlockSpec((F, F), _rep),
            pl.BlockSpec((1, F), _rep),
            pl.BlockSpec((F, F3), _rep),
        ],
        out_specs=[pl.BlockSpec((BLK, F), _row), pl.BlockSpec((BLK, F3), _row)],
        out_shape=[jax.ShapeDtypeStruct((NP, F), jnp.float32),
                   jax.ShapeDtypeStruct((NP, F3), jnp.float32)],
    )(p[0], p[1], d0, d1, h, Wl, Wr, b, W3l)


def _final_body(py0, py1, d0, d1, h, wr, b, o):
    acc = _mean(py0, py1, d0, d1)
    acc = acc + jnp.dot(h[...], wr[...], preferred_element_type=jnp.float32)
    o[...] = acc + b[...]


def _final(py, d0, d1, h, W3r, b3):
    return pl.pallas_call(
        _final_body,
        grid=(NP // BLK,),
        in_specs=[
            pl.BlockSpec((BLK, F3), _row),
            pl.BlockSpec((BLK, F3), _row),
            pl.BlockSpec((BLK, 1), _row),
            pl.BlockSpec((BLK, 1), _row),
            pl.BlockSpec((BLK, F), _row),
            pl.BlockSpec((F, F3), _rep),
            pl.BlockSpec((1, F3), _rep),
        ],
        out_specs=pl.BlockSpec((BLK, F3), _row),
        out_shape=jax.ShapeDtypeStruct((NP, F3), jnp.float32),
    )(py[0], py[1], d0, d1, h, W3r, b3)


def _pad64(w):
    return jnp.pad(w, ((0, 0), (0, F3 - w.shape[1])))


def kernel(x, edge_index, W1l, W1r, b1, W2l, W2r, b2, W3l, W3r, b3):
    src3 = edge_index[0].reshape(NW, NCHUNK, CH)
    dst3 = edge_index[1].reshape(NW, NCHUNK, CH)
    zeros2d = jnp.zeros((NP, F), jnp.float32)
    zeros64 = jnp.zeros((NP, F3), jnp.float32)
    zerosd = jnp.zeros((NP,), jnp.float32)
    xp = jnp.pad(x, ((0, NP - N), (0, 0)))

    p, degp = _sc_aggregate_deg(src3, dst3, xp, zeros2d, zerosd)
    d0 = degp[0].reshape(NP, 1)
    d1 = degp[1].reshape(NP, 1)
    h = _dense(p, d0, d1, xp, W1l, W1r, b1.reshape(1, F))

    p = _sc_aggregate(src3, dst3, h, zeros2d)
    h, y = _dense_mid(p, d0, d1, h, W2l, W2r, b2.reshape(1, F), _pad64(W3l))

    py = _sc_aggregate64(src3, dst3, y, zeros64)
    out = _final(py, d0, d1, h, _pad64(W3r),
                 jnp.pad(b3, (0, F3 - C)).reshape(1, F3))
    return out[:N, :C]


# speedup vs baseline: 1.2187x; 1.1870x over previous
import functools

import jax
import jax.numpy as jnp
from jax import lax
from jax.experimental import pallas as pl
from jax.experimental.pallas import tpu as pltpu
from jax.experimental.pallas import tpu_sc as plsc

N = 10000
E = 320000
F = 128
F3 = 128
C = 40
NP = 10240

_info = plsc.get_sparse_core_info()
NC = _info.num_cores
NS = _info.num_subcores
NW = NC * NS
EPT = E // NW
CH = 80
NCHUNK = EPT // CH
RPT = NP // NS


def _make_aggregate(width, with_deg):
    out_type = [jax.ShapeDtypeStruct((NC, NP, width), jnp.float32)]
    scratch = [
        pltpu.VMEM((2, CH), jnp.int32),
        pltpu.VMEM((2, CH), jnp.int32),
        pltpu.VMEM((CH, width), jnp.float32),
        pltpu.VMEM((CH, width), jnp.float32),
        pltpu.VMEM_SHARED((NP, width), jnp.float32),
        pltpu.SemaphoreType.DMA,
        pltpu.SemaphoreType.DMA,
        pltpu.SemaphoreType.DMA,
        pltpu.SemaphoreType.DMA,
        pltpu.SemaphoreType.DMA,
        pltpu.SemaphoreType.DMA,
    ]
    if with_deg:
        out_type.append(jax.ShapeDtypeStruct((NC, NP), jnp.float32))
        scratch += [
            pltpu.VMEM((CH,), jnp.float32),
            pltpu.VMEM_SHARED((NP,), jnp.float32),
        ]

    def body(*refs):
        if with_deg:
            (src_hbm, dst_hbm, h_hbm, zeros_hbm, zerosd_hbm,
             out_hbm, deg_hbm,
             sidx, didx, buf0, buf1, acc,
             sem0, sem1, semsi0, semsi1, semdi0, semdi1,
             ones_v, degacc) = refs
        else:
            (src_hbm, dst_hbm, h_hbm, zeros_hbm,
             out_hbm,
             sidx, didx, buf0, buf1, acc,
             sem0, sem1, semsi0, semsi1, semdi0, semdi1) = refs

        c = lax.axis_index("c")
        s = lax.axis_index("s")
        wid = c * NS + s
        r0 = s * RPT
        pltpu.sync_copy(zeros_hbm.at[pl.ds(r0, RPT)], acc.at[pl.ds(r0, RPT)])
        if with_deg:
            pltpu.sync_copy(zerosd_hbm.at[pl.ds(r0, RPT)],
                            degacc.at[pl.ds(r0, RPT)])
            for j in range(CH // 16):
                ones_v[pl.ds(j * 16, 16)] = jnp.full((16,), 1.0, jnp.float32)

        def sidx_load(i, slot, sem):
            pltpu.async_copy(src_hbm.at[wid, i], sidx.at[slot], sem)

        def sidx_wait(slot, sem):
            pltpu.make_async_copy(src_hbm.at[wid, 0], sidx.at[slot], sem).wait()

        def didx_load(i, slot, sem):
            pltpu.async_copy(dst_hbm.at[wid, i], didx.at[slot], sem)

        def didx_wait(slot, sem):
            pltpu.make_async_copy(dst_hbm.at[wid, 0], didx.at[slot], sem).wait()

        def gather(slot, buf, sem):
            pltpu.async_copy(h_hbm.at[sidx.at[slot]], buf, sem)

        def gather_wait(buf, sem):
            pltpu.make_async_copy(h_hbm.at[sidx.at[0]], buf, sem).wait()

        def scatter(buf, slot):
            pltpu.sync_copy(buf, acc.at[didx.at[slot]], add=True)
            if with_deg:
                pltpu.sync_copy(ones_v, degacc.at[didx.at[slot]], add=True)

        sidx_load(0, 0, semsi0)
        didx_load(0, 0, semdi0)
        sidx_load(1, 1, semsi1)
        didx_load(1, 1, semdi1)
        sidx_wait(0, semsi0)
        gather(0, buf0, sem0)
        plsc.subcore_barrier()

        def step(j, carry):
            i0 = 2 * j
            sidx_wait(1, semsi1)
            gather(1, buf1, sem1)
            gather_wait(buf0, sem0)
            sidx_load(i0 + 2, 0, semsi0)
            didx_wait(0, semdi0)
            scatter(buf0, 0)
            didx_load(i0 + 2, 0, semdi0)
            sidx_wait(0, semsi0)
            gather(0, buf0, sem0)
            gather_wait(buf1, sem1)

            @pl.when(i0 + 3 < NCHUNK)
            def _():
                sidx_load(i0 + 3, 1, semsi1)

            didx_wait(1, semdi1)
            scatter(buf1, 1)

            @pl.when(i0 + 3 < NCHUNK)
            def _():
                didx_load(i0 + 3, 1, semdi1)

            return carry

        lax.fori_loop(0, (NCHUNK - 1) // 2, step, 0)
        gather_wait(buf0, sem0)
        didx_wait(0, semdi0)
        scatter(buf0, 0)
        plsc.subcore_barrier()
        pltpu.sync_copy(acc.at[pl.ds(r0, RPT)], out_hbm.at[c, pl.ds(r0, RPT)])
        if with_deg:
            pltpu.sync_copy(degacc.at[pl.ds(r0, RPT)],
                            deg_hbm.at[c, pl.ds(r0, RPT)])

    return pl.kernel(
        body,
        out_type=out_type if len(out_type) > 1 else out_type[0],
        mesh=plsc.VectorSubcoreMesh(core_axis_name="c", subcore_axis_name="s"),
        scratch_types=scratch,
    )


_sc_aggregate_deg = _make_aggregate(F, True)
_sc_aggregate = _make_aggregate(F, False)
_sc_aggregate64 = _make_aggregate(F3, False)


BLK = 512
_row = lambda i: (i, 0)
_rep = lambda i: (0, 0)


def _mean(p0, p1, d0, d1):
    deg = jnp.maximum(d0[...] + d1[...], 1.0)
    return (p0[...] + p1[...]) / deg


def _dense_body(p0, p1, d0, d1, h, wl, wr, b, o, *, act):
    acc = jnp.dot(_mean(p0, p1, d0, d1), wl[...],
                  preferred_element_type=jnp.float32)
    acc = acc + jnp.dot(h[...], wr[...], preferred_element_type=jnp.float32)
    acc = acc + b[...]
    if act:
        acc = jnp.maximum(acc, 0.0)
    o[...] = acc


def _dense(p, d0, d1, h, Wl, Wr, b):
    return pl.pallas_call(
        functools.partial(_dense_body, act=True),
        grid=(NP // BLK,),
        in_specs=[
            pl.BlockSpec((BLK, F), _row),
            pl.BlockSpec((BLK, F), _row),
            pl.BlockSpec((BLK, 1), _row),
            pl.BlockSpec((BLK, 1), _row),
            pl.BlockSpec((BLK, F), _row),
            pl.BlockSpec((F, F), _rep),
            pl.BlockSpec((F, F), _rep),
            pl.BlockSpec((1, F), _rep),
        ],
        out_specs=pl.BlockSpec((BLK, F), _row),
        out_shape=jax.ShapeDtypeStruct((NP, F), jnp.float32),
    )(p[0], p[1], d0, d1, h, Wl, Wr, b)


def _dense_mid_body(p0, p1, d0, d1, h, wl, wr, b, w3l, o, oy, *, act=True):
    acc = jnp.dot(_mean(p0, p1, d0, d1), wl[...],
                  preferred_element_type=jnp.float32)
    acc = acc + jnp.dot(h[...], wr[...], preferred_element_type=jnp.float32)
    h2 = jnp.maximum(acc + b[...], 0.0)
    o[...] = h2
    oy[...] = jnp.dot(h2, w3l[...], preferred_element_type=jnp.float32)


def _dense_mid(p, d0, d1, h, Wl, Wr, b, W3l):
    return pl.pallas_call(
        _dense_mid_body,
        grid=(NP // BLK,),
        in_specs=[
            pl.BlockSpec((BLK, F), _row),
            pl.BlockSpec((BLK, F), _row),
            pl.BlockSpec((BLK, 1), _row),
            pl.BlockSpec((BLK, 1), _row),
            pl.BlockSpec((BLK, F), _row),
            pl.BlockSpec((F, F), _rep),
            pl.BlockSpec((F, F), _rep),
            pl.BlockSpec((1, F), _rep),
            pl.BlockSpec((F, F3), _rep),
        ],
        out_specs=[pl.BlockSpec((BLK, F), _row), pl.BlockSpec((BLK, F3), _row)],
        out_shape=[jax.ShapeDtypeStruct((NP, F), jnp.float32),
                   jax.ShapeDtypeStruct((NP, F3), jnp.float32)],
    )(p[0], p[1], d0, d1, h, Wl, Wr, b, W3l)


def _final_body(py0, py1, d0, d1, h, wr, b, o):
    acc = _mean(py0, py1, d0, d1)
    acc = acc + jnp.dot(h[...], wr[...], preferred_element_type=jnp.float32)
    o[...] = acc + b[...]


def _final(py, d0, d1, h, W3r, b3):
    return pl.pallas_call(
        _final_body,
        grid=(NP // BLK,),
        in_specs=[
            pl.BlockSpec((BLK, F3), _row),
            pl.BlockSpec((BLK, F3), _row),
            pl.BlockSpec((BLK, 1), _row),
            pl.BlockSpec((BLK, 1), _row),
            pl.BlockSpec((BLK, F), _row),
            pl.BlockSpec((F, F3), _rep),
            pl.BlockSpec((1, F3), _rep),
        ],
        out_specs=pl.BlockSpec((BLK, F3), _row),
        out_shape=jax.ShapeDtypeStruct((NP, F3), jnp.float32),
    )(py[0], py[1], d0, d1, h, W3r, b3)


def _pad64(w):
    return jnp.pad(w, ((0, 0), (0, F3 - w.shape[1])))


def kernel(x, edge_index, W1l, W1r, b1, W2l, W2r, b2, W3l, W3r, b3):
    src3 = edge_index[0].reshape(NW, NCHUNK, CH)
    dst3 = edge_index[1].reshape(NW, NCHUNK, CH)
    zeros2d = jnp.zeros((NP, F), jnp.float32)
    zeros64 = jnp.zeros((NP, F3), jnp.float32)
    zerosd = jnp.zeros((NP,), jnp.float32)
    xp = jnp.pad(x, ((0, NP - N), (0, 0)))

    p, degp = _sc_aggregate_deg(src3, dst3, xp, zeros2d, zerosd)
    d0 = degp[0].reshape(NP, 1)
    d1 = degp[1].reshape(NP, 1)
    h = _dense(p, d0, d1, xp, W1l, W1r, b1.reshape(1, F))

    p = _sc_aggregate(src3, dst3, h, zeros2d)
    h, y = _dense_mid(p, d0, d1, h, W2l, W2r, b2.reshape(1, F), _pad64(W3l))

    py = _sc_aggregate64(src3, dst3, y, zeros64)
    out = _final(py, d0, d1, h, _pad64(W3r),
                 jnp.pad(b3, (0, F3 - C)).reshape(1, F3))
    return out[:N, :C]
